# Initial kernel scaffold; baseline (speedup 1.0000x reference)
import jax
import jax.numpy as jnp
from jax.experimental import pallas as pl


def kernel(x, node_idx, edge_idx, W, b, w_att, b_att):
    raise NotImplementedError("write your pallas kernel here")


# speedup vs baseline: 3.3037x; 3.3037x over previous
import functools

import jax
import jax.numpy as jnp
from jax import lax
from jax.experimental import pallas as pl
from jax.experimental.pallas import tpu as pltpu
from jax.experimental.pallas import tpu_sc as plsc

N = 10000
M = 10000
NNZ = 320000
D = 128

NC = 2
NS = 16
NW = NC * NS
CH = 80
CHH = 40
B = 128
NP = 10240
DUMMY = 10000
RPW = NP // NS
CW = 16


def _seg_body(src_hbm, gidx_hbm, sidx_hbm, part, acc,
              gidx_v, sidx_v, rows0, rows1, sem0, sem1):
  c = lax.axis_index("c")
  s = lax.axis_index("s")
  wid = s * NC + c

  zero16 = jnp.zeros((16,), jnp.float32)

  def _zrow(i, _):
    for k in range(D // 16):
      rows0[i, pl.ds(k * 16, 16)] = zero16
    return 0

  lax.fori_loop(0, B, _zrow, 0, unroll=False)

  r0 = s * RPW

  def _zslice(t, _):
    pltpu.sync_copy(rows0, acc.at[pl.ds(r0 + t * B, B)])
    return 0

  lax.fori_loop(0, RPW // B, _zslice, 0, unroll=False)

  plsc.subcore_barrier()

  def _chunk(j, rows, sem):
    pltpu.make_async_copy(src_hbm.at[gidx_v.at[j]], rows, sem).wait()
    pltpu.sync_copy(rows, acc.at[sidx_v.at[j]], add=True)

    @pl.when(j + 2 < CHH)
    def _():
      pltpu.async_copy(src_hbm.at[gidx_v.at[j + 2]], rows, sem)

  def _pair(k, _):
    _chunk(2 * k, rows0, sem0)
    _chunk(2 * k + 1, rows1, sem1)
    return 0

  for h in range(CH // CHH):
    pltpu.sync_copy(gidx_hbm.at[wid, pl.ds(h * CHH, CHH)], gidx_v)
    pltpu.sync_copy(sidx_hbm.at[wid, pl.ds(h * CHH, CHH)], sidx_v)
    pltpu.async_copy(src_hbm.at[gidx_v.at[0]], rows0, sem0)
    pltpu.async_copy(src_hbm.at[gidx_v.at[1]], rows1, sem1)
    lax.fori_loop(0, CHH // 2, _pair, 0, unroll=False)

  plsc.subcore_barrier()

  def _pslice(t, _):
    sl = pl.ds(r0 + t * B, B)
    pltpu.sync_copy(acc.at[sl], part.at[c, sl])
    return 0

  lax.fori_loop(0, RPW // B, _pslice, 0, unroll=False)


def _ones_body(sidx_hbm, part, acc, sidx_v, rows0, rows1):
  c = lax.axis_index("c")
  s = lax.axis_index("s")
  wid = s * NC + c

  zero16 = jnp.zeros((16,), jnp.float32)
  one16 = jnp.full((16,), 1.0, jnp.float32)

  def _frow(i, _):
    for k in range(D // 16):
      rows0[i, pl.ds(k * 16, 16)] = zero16
      rows1[i, pl.ds(k * 16, 16)] = one16
    return 0

  lax.fori_loop(0, B, _frow, 0, unroll=False)

  r0 = s * RPW

  def _zslice(t, _):
    pltpu.sync_copy(rows0, acc.at[pl.ds(r0 + t * B, B)])
    return 0

  lax.fori_loop(0, RPW // B, _zslice, 0, unroll=False)

  plsc.subcore_barrier()

  def _cchunk(j, _):
    pltpu.sync_copy(rows1, acc.at[sidx_v.at[j]], add=True)
    return 0

  for h in range(CH // CHH):
    pltpu.sync_copy(sidx_hbm.at[wid, pl.ds(h * CHH, CHH)], sidx_v)
    lax.fori_loop(0, CHH, _cchunk, 0, unroll=False)

  plsc.subcore_barrier()

  def _pslice(t, _):
    sl = pl.ds(r0 + t * B, B)
    pltpu.sync_copy(acc.at[sl], part.at[c, sl])
    return 0

  lax.fori_loop(0, RPW // B, _pslice, 0, unroll=False)


@functools.cache
def _sc_mesh():
  return plsc.VectorSubcoreMesh(
      core_axis_name="c", subcore_axis_name="s",
      num_cores=NC, num_subcores=NS)


@functools.cache
def _make_seg_kernel():
  return pl.kernel(
      _seg_body,
      out_type=jax.ShapeDtypeStruct((NC, NP, D), jnp.float32),
      mesh=_sc_mesh(),
      scratch_types=[
          pltpu.VMEM_SHARED((NP, D), jnp.float32),
          pltpu.VMEM((CHH, B), jnp.int32),
          pltpu.VMEM((CHH, B), jnp.int32),
          pltpu.VMEM((B, D), jnp.float32),
          pltpu.VMEM((B, D), jnp.float32),
          pltpu.SemaphoreType.DMA,
          pltpu.SemaphoreType.DMA,
      ],
  )


@functools.cache
def _make_ones_kernel():
  return pl.kernel(
      _ones_body,
      out_type=jax.ShapeDtypeStruct((NC, NP, D), jnp.float32),
      mesh=_sc_mesh(),
      scratch_types=[
          pltpu.VMEM_SHARED((NP, D), jnp.float32),
          pltpu.VMEM((CHH, B), jnp.int32),
          pltpu.VMEM((B, D), jnp.float32),
          pltpu.VMEM((B, D), jnp.float32),
      ],
  )


_ROWBLK = 1024
_GRID = NP // _ROWBLK


def _mm_body(x_ref, w_ref, b_ref, o_ref):
  o_ref[...] = (
      jnp.dot(x_ref[...], w_ref[...], preferred_element_type=jnp.float32)
      + b_ref[...]
  )


_matmul = pl.pallas_call(
    _mm_body,
    grid=(_GRID,),
    in_specs=[
        pl.BlockSpec((_ROWBLK, D), lambda i: (i, 0)),
        pl.BlockSpec((D, D), lambda i: (0, 0)),
        pl.BlockSpec((1, D), lambda i: (0, 0)),
    ],
    out_specs=pl.BlockSpec((_ROWBLK, D), lambda i: (i, 0)),
    out_shape=jax.ShapeDtypeStruct((NP, D), jnp.float32),
)


def _combine_body(p_ref, c_ref, o_ref):
  ssum = p_ref[0] + p_ref[1]
  cnt = c_ref[0, :, :1] + c_ref[1, :, :1]
  o_ref[...] = ssum / jnp.maximum(cnt, 1.0)


_combine = pl.pallas_call(
    _combine_body,
    grid=(_GRID,),
    in_specs=[
        pl.BlockSpec((NC, _ROWBLK, D), lambda i: (0, i, 0)),
        pl.BlockSpec((NC, _ROWBLK, D), lambda i: (0, i, 0)),
    ],
    out_specs=pl.BlockSpec((_ROWBLK, D), lambda i: (i, 0)),
    out_shape=jax.ShapeDtypeStruct((NP, D), jnp.float32),
)


def _combine_att_body(p_ref, c_ref, wa_ref, ba_ref, o_ref):
  ssum = p_ref[0] + p_ref[1]
  cnt = c_ref[0, :, :1] + c_ref[1, :, :1]
  ef = ssum / jnp.maximum(cnt, 1.0)
  z = jnp.dot(ef, wa_ref[...], preferred_element_type=jnp.float32) + ba_ref[...]
  ew = jax.nn.sigmoid(z)
  o_ref[...] = ew * ef


_combine_att = pl.pallas_call(
    _combine_att_body,
    grid=(_GRID,),
    in_specs=[
        pl.BlockSpec((NC, _ROWBLK, D), lambda i: (0, i, 0)),
        pl.BlockSpec((NC, _ROWBLK, D), lambda i: (0, i, 0)),
        pl.BlockSpec((D, 1), lambda i: (0, 0)),
        pl.BlockSpec((1, 1), lambda i: (0, 0)),
    ],
    out_specs=pl.BlockSpec((_ROWBLK, D), lambda i: (i, 0)),
    out_shape=jax.ShapeDtypeStruct((NP, D), jnp.float32),
)


def kernel(x, node_idx, edge_idx, W, b, w_att, b_att):
  pad = NW * CH * B - NNZ
  node_slab = jnp.concatenate(
      [node_idx.astype(jnp.int32),
       jnp.full((pad,), DUMMY, jnp.int32)]).reshape(NW, CH, B)
  edge_slab = jnp.concatenate(
      [edge_idx.astype(jnp.int32),
       jnp.full((pad,), DUMMY, jnp.int32)]).reshape(NW, CH, B)
  x_pad = jnp.zeros((NP, D), x.dtype).at[:N].set(x)

  x1 = _matmul(x_pad, W, b.reshape(1, D))

  seg_plain = _make_seg_kernel()
  ones_seg = _make_ones_kernel()

  ncnt = ones_seg(node_slab)
  ecnt = ones_seg(edge_slab)
  pa = seg_plain(x1, node_slab, edge_slab)
  e_feat = _combine(pa, ecnt)

  pb = seg_plain(e_feat, edge_slab, node_slab)
  x2 = _combine(pb, ncnt)

  pc = seg_plain(x2, node_slab, edge_slab)
  g = _combine_att(pc, ecnt, w_att, b_att.reshape(1, 1))

  pd = seg_plain(g, edge_slab, node_slab)
  x3 = _combine(pd, ncnt)

  return x3[:N]
